# baseline (speedup 1.0000x reference)
import jax
import jax.numpy as jnp
from jax.experimental import pallas as pl
from jax.experimental.pallas import tpu as pltpu


def _fused_kernel(x_hbm, w_ref, a_ref, o_ref, t_ref, xbuf, sems):
    n_rows = x_hbm.shape[0]
    bc = xbuf.shape[1]
    chunks = n_rows // bc

    @pl.when(pl.program_id(0) == 0)
    def _compute_t():
        w = w_ref[...].astype(jnp.bfloat16)

        def start(c):
            pltpu.make_async_copy(
                x_hbm.at[pl.ds(c * bc, bc), :], xbuf.at[c % 2],
                sems.at[c % 2]).start()

        start(0)
        for c in range(chunks):
            if c + 1 < chunks:
                start(c + 1)
            pltpu.make_async_copy(
                x_hbm.at[pl.ds(c * bc, bc), :], xbuf.at[c % 2],
                sems.at[c % 2]).wait()
            t = jax.lax.dot_general(
                xbuf[c % 2].astype(jnp.bfloat16), w,
                (((1,), (0,)), ((), ())),
                preferred_element_type=jnp.float32)
            t_ref[pl.ds(c * bc, bc), :] = t.astype(jnp.bfloat16)

    acc = jax.lax.dot_general(
        a_ref[...].astype(jnp.bfloat16), t_ref[...],
        (((1,), (0,)), ((), ())),
        preferred_element_type=jnp.float32)
    o_ref[...] = jnp.maximum(acc, 0.0)


def kernel(x_source, neighborhood, W1, W2, W3):
    n, d_in = x_source.shape
    d_out = W1.shape[1]
    bi = min(512, n)
    bc = min(512, n)

    out = pl.pallas_call(
        _fused_kernel,
        grid=(n // bi,),
        in_specs=[pl.BlockSpec(memory_space=pltpu.MemorySpace.HBM),
                  pl.BlockSpec((d_in, d_out), lambda i: (0, 0)),
                  pl.BlockSpec((bi, n), lambda i: (i, 0))],
        out_specs=pl.BlockSpec((bi, d_out), lambda i: (i, 0)),
        out_shape=jax.ShapeDtypeStruct((n, d_out), jnp.float32),
        scratch_shapes=[pltpu.VMEM((n, d_out), jnp.bfloat16),
                        pltpu.VMEM((2, bc, d_in), jnp.float32),
                        pltpu.SemaphoreType.DMA((2,))],
    )(x_source, W1, neighborhood)
    return out


# speedup vs baseline: 1.0500x; 1.0500x over previous
import jax
import jax.numpy as jnp
from jax.experimental import pallas as pl
from jax.experimental.pallas import tpu as pltpu


def _fused_kernel(x_hbm, w_ref, a_ref, o_ref, t_ref, xbuf, sems):
    n_rows = x_hbm.shape[0]
    bc = xbuf.shape[1]
    chunks = n_rows // bc

    @pl.when(pl.program_id(0) == 0)
    def _compute_t():
        w = w_ref[...].astype(jnp.bfloat16)

        def start(c):
            pltpu.make_async_copy(
                x_hbm.at[pl.ds(c * bc, bc), :], xbuf.at[c % 2],
                sems.at[c % 2]).start()

        start(0)
        for c in range(chunks):
            if c + 1 < chunks:
                start(c + 1)
            pltpu.make_async_copy(
                x_hbm.at[pl.ds(c * bc, bc), :], xbuf.at[c % 2],
                sems.at[c % 2]).wait()
            t = jax.lax.dot_general(
                xbuf[c % 2].astype(jnp.bfloat16), w,
                (((1,), (0,)), ((), ())),
                preferred_element_type=jnp.float32)
            t_ref[pl.ds(c * bc, bc), :] = t.astype(jnp.bfloat16)

    acc = jax.lax.dot_general(
        a_ref[...].astype(jnp.bfloat16), t_ref[...],
        (((1,), (0,)), ((), ())),
        preferred_element_type=jnp.float32)
    o_ref[...] = jnp.maximum(acc, 0.0)


def kernel(x_source, neighborhood, W1, W2, W3):
    n, d_in = x_source.shape
    d_out = W1.shape[1]
    bi = min(512, n)
    bc = min(2048, n)

    out = pl.pallas_call(
        _fused_kernel,
        grid=(n // bi,),
        in_specs=[pl.BlockSpec(memory_space=pltpu.MemorySpace.HBM),
                  pl.BlockSpec((d_in, d_out), lambda i: (0, 0)),
                  pl.BlockSpec((bi, n), lambda i: (i, 0))],
        out_specs=pl.BlockSpec((bi, d_out), lambda i: (i, 0)),
        out_shape=jax.ShapeDtypeStruct((n, d_out), jnp.float32),
        scratch_shapes=[pltpu.VMEM((n, d_out), jnp.bfloat16),
                        pltpu.VMEM((2, bc, d_in), jnp.float32),
                        pltpu.SemaphoreType.DMA((2,))],
    )(x_source, W1, neighborhood)
    return out


# speedup vs baseline: 1.0563x; 1.0060x over previous
import jax
import jax.numpy as jnp
from jax.experimental import pallas as pl
from jax.experimental.pallas import tpu as pltpu


def _fused_kernel(x_hbm, w_ref, a_ref, o_ref, t_ref, xbuf, sems):
    n_rows = x_hbm.shape[0]
    bc = xbuf.shape[1]
    chunks = n_rows // bc

    slots = xbuf.shape[0]

    @pl.when(pl.program_id(0) == 0)
    def _compute_t():
        w = w_ref[...].astype(jnp.bfloat16)

        def start(c):
            pltpu.make_async_copy(
                x_hbm.at[pl.ds(c * bc, bc), :], xbuf.at[c % slots],
                sems.at[c % slots]).start()

        for c in range(min(slots - 1, chunks)):
            start(c)
        for c in range(chunks):
            if c + slots - 1 < chunks:
                start(c + slots - 1)
            pltpu.make_async_copy(
                x_hbm.at[pl.ds(c * bc, bc), :], xbuf.at[c % slots],
                sems.at[c % slots]).wait()
            t = jax.lax.dot_general(
                xbuf[c % slots].astype(jnp.bfloat16), w,
                (((1,), (0,)), ((), ())),
                preferred_element_type=jnp.float32)
            t_ref[pl.ds(c * bc, bc), :] = t.astype(jnp.bfloat16)

    acc = jax.lax.dot_general(
        a_ref[...].astype(jnp.bfloat16), t_ref[...],
        (((1,), (0,)), ((), ())),
        preferred_element_type=jnp.float32)
    o_ref[...] = jnp.maximum(acc, 0.0)


def kernel(x_source, neighborhood, W1, W2, W3):
    n, d_in = x_source.shape
    d_out = W1.shape[1]
    bi = min(512, n)
    bc = min(2048, n)

    out = pl.pallas_call(
        _fused_kernel,
        grid=(n // bi,),
        in_specs=[pl.BlockSpec(memory_space=pltpu.MemorySpace.HBM),
                  pl.BlockSpec((d_in, d_out), lambda i: (0, 0)),
                  pl.BlockSpec((bi, n), lambda i: (i, 0))],
        out_specs=pl.BlockSpec((bi, d_out), lambda i: (i, 0)),
        out_shape=jax.ShapeDtypeStruct((n, d_out), jnp.float32),
        scratch_shapes=[pltpu.VMEM((n, d_out), jnp.bfloat16),
                        pltpu.VMEM((3, bc, d_in), jnp.float32),
                        pltpu.SemaphoreType.DMA((3,))],
    )(x_source, W1, neighborhood)
    return out
